# baseline (speedup 1.0000x reference)
import functools

import jax
import jax.numpy as jnp
from jax import lax
from jax.experimental import pallas as pl
from jax.experimental.pallas import tpu as pltpu
from jax.experimental.pallas import tpu_sc as plsc

_K = 32
_EMB_PAD = 16
_N = 8192
_NV = _N // 16
_BIG = 1 << 30
_TOK_W = 12
_NREG = 8
_RCH = _NV // _NREG
_RSTRIDE = _RCH * 16 + 16
_FASTI = _NREG * _RSTRIDE
_SERI = _FASTI + 256


def _sc_call(rssi_flat, emb_pad, rank_flat, n_rows):
    info = plsc.get_sparse_core_info()
    nc, ns = info.num_cores, info.num_subcores
    nw = nc * ns
    rpw = n_rows // nw

    mesh = plsc.VectorSubcoreMesh(core_axis_name="c", subcore_axis_name="s")

    @functools.partial(
        pl.kernel,
        mesh=mesh,
        compiler_params=pltpu.CompilerParams(use_tc_tiling_on_sc=False,
                                             needs_layout_passes=False),
        out_type=jax.ShapeDtypeStruct((n_rows * _K * _TOK_W,), jnp.float32),
        scratch_types=[
            pltpu.VMEM((_N,), jnp.float32),
            pltpu.VMEM((_N,), jnp.float32),
            pltpu.VMEM((_N + 16,), jnp.float32),
            pltpu.VMEM((_SERI + _N + 16,), jnp.int32),
            pltpu.VMEM((_K,), jnp.float32),
            pltpu.VMEM((_K,), jnp.int32),
            pltpu.VMEM((_K, _EMB_PAD), jnp.float32),
            pltpu.VMEM((_K * _TOK_W,), jnp.float32),
            pltpu.VMEM((_K,), jnp.float32),
            pltpu.SemaphoreType.DMA,
        ],
    )
    def k(rssi_hbm, emb_hbm, rank_hbm, out_hbm,
          row_a, row_b, cand_v, cand_i, topv, topi, embbuf, tokbuf, rankbuf,
          sem):
        wid = lax.axis_index("s") * nc + lax.axis_index("c")
        base_row = wid * rpw
        iota16 = lax.iota(jnp.int32, 16)
        pltpu.sync_copy(rank_hbm, rankbuf)

        lane0 = iota16 == 0

        def emit_top(j, m, b):
            jsplat = jnp.full((16,), j, jnp.int32)
            plsc.store_scatter(topv, [jsplat],
                               jnp.broadcast_to(m, (16,)), mask=lane0)
            plsc.store_scatter(topi, [jsplat],
                               jnp.broadcast_to(b, (16,)), mask=lane0)

        def process_row(row, cur, prev, is_t0, t_spec):
            neg1 = jnp.full((16,), -1.0, jnp.float32)

            def run_fast(get_vreg):
                summ = jnp.full((16,), -1.0, jnp.float32)
                for v in range(16):
                    cv, ci, valid = get_vreg(v)
                    cv = jnp.where(valid, cv, -1.0)
                    cand_v[pl.ds(v * 16, 16)] = cv
                    cand_i[pl.ds(_FASTI + v * 16, 16)] = ci
                    summ = jnp.where(iota16 == v, jnp.max(cv), summ)

                def ext(j, summ):
                    m = jnp.max(summ)
                    bv = plsc.all_reduce_ffs(summ == m)[0]
                    sl = pl.ds(bv * 16, 16)
                    cv = cand_v[sl]
                    l0 = plsc.all_reduce_ffs(cv == m)
                    ci = cand_i[pl.ds(_FASTI + bv * 16, 16)]
                    b = ci.at[l0].get(mode="promise_in_bounds")
                    emit_top(j, jnp.full((16,), m), b)
                    cv = jnp.where(iota16 == l0, -1.0, cv)
                    cand_v[sl] = cv
                    return jnp.where(iota16 == bv, jnp.max(cv), summ)

                lax.fori_loop(0, _K, ext, summ)

            def collect_serial(thr):
                @plsc.parallel_loop(0, _NV, unroll=8, carry=jnp.int32(0))
                def collect(c, cnt):
                    msk = cur[pl.ds(c * 16, 16)] >= thr
                    plsc.store_compressed(
                        cand_i.at[pl.ds(_SERI + cnt, 16)],
                        iota16 + c * 16, mask=msk)
                    return cnt + plsc.all_reduce_population_count(msk)[0]

                return collect

            def exact_thr():
                @plsc.parallel_loop(0, _NV // 2, unroll=8, carry=(neg1, neg1))
                def amax(c, ms):
                    m1, m2 = ms
                    return (jnp.maximum(m1, cur[pl.ds(c * 16, 16)]),
                            jnp.maximum(m2,
                                        cur[pl.ds((c + _NV // 2) * 16, 16)]))

                m1, m2 = amax
                return jnp.min(jnp.minimum(m1, m2))

            zero8 = (jnp.int32(0),) * _NREG

            @plsc.parallel_loop(0, _RCH, unroll=2, carry=zero8)
            def rcollect(c, cnts):
                out = []
                for r in range(_NREG):
                    ch = r * _RCH + c
                    msk = cur[pl.ds(ch * 16, 16)] >= t_spec
                    plsc.store_compressed(
                        cand_i.at[pl.ds(r * _RSTRIDE + cnts[r], 16)],
                        iota16 + ch * 16, mask=msk)
                    out.append(
                        cnts[r] + plsc.all_reduce_population_count(msk)[0])
                return tuple(out)

            cnts = rcollect
            total = cnts[0]
            fast_ok = cnts[0] <= 32
            for r in range(1, _NREG):
                total = total + cnts[r]
                fast_ok = fast_ok & (cnts[r] <= 32)
            fast_ok = fast_ok & (total >= _K)

            @pl.when(fast_ok)
            def _spec_path():
                def get_region(v):
                    r, o = divmod(v, 2)
                    ci = cand_i[pl.ds(r * _RSTRIDE + o * 16, 16)]
                    valid = iota16 + o * 16 < cnts[r]
                    cv = plsc.load_gather(cur, [ci], mask=valid)
                    return cv, ci, valid

                run_fast(get_region)

            @pl.when(jnp.logical_not(fast_ok))
            def _fallback():
                cnt = collect_serial(exact_thr())

                @pl.when(cnt <= 256)
                def _fast():
                    def get_serial(v):
                        ci = cand_i[pl.ds(_SERI + v * 16, 16)]
                        valid = iota16 + v * 16 < cnt
                        cv = plsc.load_gather(cur, [ci], mask=valid)
                        return cv, ci, valid

                    run_fast(get_serial)

                @pl.when(cnt > 256)
                def _slow():
                    nv = (cnt + 15) // 16

                    def mat(v, _):
                        sl = pl.ds(v * 16, 16)
                        valid = iota16 + v * 16 < cnt
                        ci = cand_i[pl.ds(_SERI + v * 16, 16)]
                        cv = plsc.load_gather(cur, [ci], mask=valid)
                        cand_v[sl] = jnp.where(valid, cv, -1.0)
                        return 0

                    lax.fori_loop(0, nv, mat, 0)

                    def extract(j, _):
                        def p1(v, m):
                            return jnp.maximum(m, cand_v[pl.ds(v * 16, 16)])

                        m = jnp.max(lax.fori_loop(0, nv, p1, neg1))

                        def p2(v, b):
                            cv = cand_v[pl.ds(v * 16, 16)]
                            ci = cand_i[pl.ds(_SERI + v * 16, 16)]
                            return jnp.minimum(
                                b, jnp.where(cv == m, ci, _BIG))

                        b = jnp.min(lax.fori_loop(
                            0, nv, p2, jnp.full((16,), _BIG, jnp.int32)))
                        emit_top(j, m, b)

                        def p3(v, _):
                            sl = pl.ds(v * 16, 16)
                            cv = cand_v[sl]
                            ci = cand_i[pl.ds(_SERI + v * 16, 16)]
                            cand_v[sl] = jnp.where(
                                (cv == m) & (ci == b), -1.0, cv)
                            return 0

                        lax.fori_loop(0, nv, p3, 0)
                        return 0

                    lax.fori_loop(0, _K, extract, 0)

            dma = pltpu.async_copy(emb_hbm.at[topi], embbuf, sem)
            for h in range(2):
                sl = pl.ds(h * 16, 16)
                tv = topv[sl]
                ti = topi[sl]
                pv = jnp.where(is_t0, tv, plsc.load_gather(prev, [ti]))
                delta = tv - pv
                isnew = jnp.where((pv <= 1e-6) & (tv > 1e-6), 1.0, 0.0)
                rk = rankbuf[sl]
                tgt = (iota16 + h * 16) * _TOK_W
                plsc.store_scatter(tokbuf, [tgt + 8], tv)
                plsc.store_scatter(tokbuf, [tgt + 9], delta)
                plsc.store_scatter(tokbuf, [tgt + 10], rk)
                plsc.store_scatter(tokbuf, [tgt + 11], isnew)
            dma.wait()
            for h in range(2):
                tok = iota16 + h * 16
                tgt = tok * _TOK_W
                for d in range(8):
                    ev = plsc.load_gather(
                        embbuf, [tok, jnp.full((16,), d, jnp.int32)])
                    plsc.store_scatter(tokbuf, [tgt + d], ev)
            pltpu.sync_copy(tokbuf,
                            out_hbm.at[pl.ds(row * (_K * _TOK_W),
                                             _K * _TOK_W)])
            return 2.0 * topv[pl.ds(_K - 16, 16)][15] - 1.0

        def pair(i, t_spec):
            r0 = base_row + 2 * i
            pltpu.sync_copy(rssi_hbm.at[pl.ds(r0 * _N, _N)], row_a)
            t_spec = process_row(r0, row_a, row_b, (2 * i) % 32 == 0, t_spec)
            r1 = r0 + 1
            pltpu.sync_copy(rssi_hbm.at[pl.ds(r1 * _N, _N)], row_b)
            return process_row(r1, row_b, row_a, False, t_spec)

        lax.fori_loop(0, rpw // 2, pair, jnp.float32(2.0))

    return k(rssi_flat, emb_pad, rank_flat)


def kernel(rssi_seq, ap_emb, rank_template):
    b_dim, t_dim, _ = rssi_seq.shape
    emb_dim = ap_emb.shape[1]
    n_rows = b_dim * t_dim
    emb_pad = jnp.pad(ap_emb, ((0, 0), (0, _EMB_PAD - emb_dim)))
    out = _sc_call(rssi_seq.reshape(-1), emb_pad,
                   rank_template.reshape(-1), n_rows)
    return out.reshape(b_dim, t_dim, _K, _TOK_W)


# speedup vs baseline: 1.2959x; 1.2959x over previous
import functools

import jax
import jax.numpy as jnp
from jax import lax
from jax.experimental import pallas as pl
from jax.experimental.pallas import tpu as pltpu
from jax.experimental.pallas import tpu_sc as plsc

_K = 32
_EMB_PAD = 16
_N = 8192
_NV = _N // 16
_BIG = 1 << 30
_TOK_W = 12
_FASTI = 0
_SERI = 256


def _sc_call(rssi_flat, emb_pad, rank_flat, n_rows):
    info = plsc.get_sparse_core_info()
    nc, ns = info.num_cores, info.num_subcores
    nw = nc * ns
    rpw = n_rows // nw

    mesh = plsc.VectorSubcoreMesh(core_axis_name="c", subcore_axis_name="s")

    @functools.partial(
        pl.kernel,
        mesh=mesh,
        compiler_params=pltpu.CompilerParams(use_tc_tiling_on_sc=False,
                                             needs_layout_passes=False),
        out_type=jax.ShapeDtypeStruct((n_rows * _K * _TOK_W,), jnp.float32),
        scratch_types=[
            pltpu.VMEM((_N,), jnp.float32),
            pltpu.VMEM((_N,), jnp.float32),
            pltpu.VMEM((_N,), jnp.float32),
            pltpu.VMEM((_N,), jnp.float32),
            pltpu.VMEM((_N + 16,), jnp.float32),
            pltpu.VMEM((_SERI + _N + 16,), jnp.int32),
            pltpu.VMEM((_K,), jnp.float32),
            pltpu.VMEM((_K,), jnp.int32),
            pltpu.VMEM((_K, _EMB_PAD), jnp.float32),
            pltpu.VMEM((_K * _TOK_W,), jnp.float32),
            pltpu.VMEM((_K,), jnp.float32),
            pltpu.SemaphoreType.DMA,
            pltpu.SemaphoreType.DMA,
            pltpu.SemaphoreType.DMA,
            pltpu.SemaphoreType.DMA,
            pltpu.SemaphoreType.DMA,
        ],
    )
    def k(rssi_hbm, emb_hbm, rank_hbm, out_hbm,
          row_0, row_1, row_2, row_3, cand_v, cand_i, topv, topi, embbuf,
          tokbuf, rankbuf, sem, rsem_0, rsem_1, rsem_2, rsem_3):
        wid = lax.axis_index("s") * nc + lax.axis_index("c")
        base_row = wid * rpw
        iota16 = lax.iota(jnp.int32, 16)
        pltpu.sync_copy(rank_hbm, rankbuf)

        lane0 = iota16 == 0

        def emit_top(j, m, b):
            jsplat = jnp.full((16,), j, jnp.int32)
            plsc.store_scatter(topv, [jsplat],
                               jnp.broadcast_to(m, (16,)), mask=lane0)
            plsc.store_scatter(topi, [jsplat],
                               jnp.broadcast_to(b, (16,)), mask=lane0)

        def process_row(row, cur, prev, is_t0, t_spec):
            neg1 = jnp.full((16,), -1.0, jnp.float32)

            def run_fast(get_vreg):
                summ = jnp.full((16,), -1.0, jnp.float32)
                for v in range(16):
                    cv, ci, valid = get_vreg(v)
                    cv = jnp.where(valid, cv, -1.0)
                    cand_v[pl.ds(v * 16, 16)] = cv
                    cand_i[pl.ds(_FASTI + v * 16, 16)] = ci
                    summ = jnp.where(iota16 == v, jnp.max(cv), summ)

                def ext(j, summ):
                    m = jnp.max(summ)
                    bv = plsc.all_reduce_ffs(summ == m)[0]
                    sl = pl.ds(bv * 16, 16)
                    cv = cand_v[sl]
                    l0 = plsc.all_reduce_ffs(cv == m)
                    ci = cand_i[pl.ds(_FASTI + bv * 16, 16)]
                    b = ci.at[l0].get(mode="promise_in_bounds")
                    emit_top(j, jnp.full((16,), m), b)
                    cv = jnp.where(iota16 == l0, -1.0, cv)
                    cand_v[sl] = cv
                    return jnp.where(iota16 == bv, jnp.max(cv), summ)

                lax.fori_loop(0, _K, ext, summ)

            def collect_serial(thr):
                @plsc.parallel_loop(0, _NV, unroll=8, carry=jnp.int32(0))
                def collect(c, cnt):
                    msk = cur[pl.ds(c * 16, 16)] >= thr
                    plsc.store_compressed(
                        cand_i.at[pl.ds(_SERI + cnt, 16)],
                        iota16 + c * 16, mask=msk)
                    return cnt + plsc.all_reduce_population_count(msk)[0]

                return collect

            def exact_thr():
                @plsc.parallel_loop(0, _NV // 2, unroll=8, carry=(neg1, neg1))
                def amax(c, ms):
                    m1, m2 = ms
                    return (jnp.maximum(m1, cur[pl.ds(c * 16, 16)]),
                            jnp.maximum(m2,
                                        cur[pl.ds((c + _NV // 2) * 16, 16)]))

                m1, m2 = amax
                return jnp.min(jnp.minimum(m1, m2))

            cnt0 = collect_serial(t_spec)
            fast_ok = (cnt0 >= _K) & (cnt0 <= 256)

            @pl.when(fast_ok)
            def _spec_path():
                def get_spec(v):
                    ci = cand_i[pl.ds(_SERI + v * 16, 16)]
                    valid = iota16 + v * 16 < cnt0
                    cv = plsc.load_gather(cur, [ci], mask=valid)
                    return cv, ci, valid

                run_fast(get_spec)

            @pl.when(jnp.logical_not(fast_ok))
            def _fallback():
                cnt = collect_serial(exact_thr())

                @pl.when(cnt <= 256)
                def _fast():
                    def get_serial(v):
                        ci = cand_i[pl.ds(_SERI + v * 16, 16)]
                        valid = iota16 + v * 16 < cnt
                        cv = plsc.load_gather(cur, [ci], mask=valid)
                        return cv, ci, valid

                    run_fast(get_serial)

                @pl.when(cnt > 256)
                def _slow():
                    nv = (cnt + 15) // 16

                    def mat(v, _):
                        sl = pl.ds(v * 16, 16)
                        valid = iota16 + v * 16 < cnt
                        ci = cand_i[pl.ds(_SERI + v * 16, 16)]
                        cv = plsc.load_gather(cur, [ci], mask=valid)
                        cand_v[sl] = jnp.where(valid, cv, -1.0)
                        return 0

                    lax.fori_loop(0, nv, mat, 0)

                    def extract(j, _):
                        def p1(v, m):
                            return jnp.maximum(m, cand_v[pl.ds(v * 16, 16)])

                        m = jnp.max(lax.fori_loop(0, nv, p1, neg1))

                        def p2(v, b):
                            cv = cand_v[pl.ds(v * 16, 16)]
                            ci = cand_i[pl.ds(_SERI + v * 16, 16)]
                            return jnp.minimum(
                                b, jnp.where(cv == m, ci, _BIG))

                        b = jnp.min(lax.fori_loop(
                            0, nv, p2, jnp.full((16,), _BIG, jnp.int32)))
                        emit_top(j, m, b)

                        def p3(v, _):
                            sl = pl.ds(v * 16, 16)
                            cv = cand_v[sl]
                            ci = cand_i[pl.ds(_SERI + v * 16, 16)]
                            cand_v[sl] = jnp.where(
                                (cv == m) & (ci == b), -1.0, cv)
                            return 0

                        lax.fori_loop(0, nv, p3, 0)
                        return 0

                    lax.fori_loop(0, _K, extract, 0)

            dma = pltpu.async_copy(emb_hbm.at[topi], embbuf, sem)
            for h in range(2):
                sl = pl.ds(h * 16, 16)
                tv = topv[sl]
                ti = topi[sl]
                pv = jnp.where(is_t0, tv, plsc.load_gather(prev, [ti]))
                delta = tv - pv
                isnew = jnp.where((pv <= 1e-6) & (tv > 1e-6), 1.0, 0.0)
                rk = rankbuf[sl]
                tgt = (iota16 + h * 16) * _TOK_W
                plsc.store_scatter(tokbuf, [tgt + 8], tv)
                plsc.store_scatter(tokbuf, [tgt + 9], delta)
                plsc.store_scatter(tokbuf, [tgt + 10], rk)
                plsc.store_scatter(tokbuf, [tgt + 11], isnew)
            dma.wait()
            for h in range(2):
                tok = iota16 + h * 16
                tgt = tok * _TOK_W
                for d in range(8):
                    ev = plsc.load_gather(
                        embbuf, [tok, jnp.full((16,), d, jnp.int32)])
                    plsc.store_scatter(tokbuf, [tgt + d], ev)
            pltpu.sync_copy(tokbuf,
                            out_hbm.at[pl.ds(row * (_K * _TOK_W),
                                             _K * _TOK_W)])
            return 2.0 * topv[pl.ds(_K - 16, 16)][15] - 1.0

        rows = [row_0, row_1, row_2, row_3]
        rsems = [rsem_0, rsem_1, rsem_2, rsem_3]

        def issue_row(r, b):
            rr = jnp.minimum(r, n_rows - 1)
            pltpu.async_copy(rssi_hbm.at[pl.ds(rr * _N, _N)], rows[b],
                             rsems[b])

        def wait_row(b):
            pltpu.make_async_copy(rssi_hbm.at[pl.ds(0, _N)], rows[b],
                                  rsems[b]).wait()

        issue_row(base_row, 0)
        issue_row(base_row + 1, 1)

        def quad(j, t_spec):
            r0 = base_row + 4 * j
            for t in range(4):
                issue_row(r0 + t + 2, (t + 2) % 4)
                wait_row(t)
                t_spec = process_row(r0 + t, rows[t], rows[(t + 3) % 4],
                                     (t == 0) & (j % 8 == 0), t_spec)
            return t_spec

        lax.fori_loop(0, rpw // 4, quad, jnp.float32(2.0))
        wait_row(0)
        wait_row(1)

    return k(rssi_flat, emb_pad, rank_flat)


def kernel(rssi_seq, ap_emb, rank_template):
    b_dim, t_dim, _ = rssi_seq.shape
    emb_dim = ap_emb.shape[1]
    n_rows = b_dim * t_dim
    emb_pad = jnp.pad(ap_emb, ((0, 0), (0, _EMB_PAD - emb_dim)))
    out = _sc_call(rssi_seq.reshape(-1), emb_pad,
                   rank_template.reshape(-1), n_rows)
    return out.reshape(b_dim, t_dim, _K, _TOK_W)


# speedup vs baseline: 1.3285x; 1.0252x over previous
import functools

import jax
import jax.numpy as jnp
from jax import lax
from jax.experimental import pallas as pl
from jax.experimental.pallas import tpu as pltpu
from jax.experimental.pallas import tpu_sc as plsc

_K = 32
_EMB_PAD = 16
_N = 8192
_NV = _N // 16
_BIG = 1 << 30
_TOK_W = 12
_FASTI = 0
_SERI = 256


def _sc_call(rssi_flat, emb_pad, rank_flat, n_rows):
    info = plsc.get_sparse_core_info()
    nc, ns = info.num_cores, info.num_subcores
    nw = nc * ns
    rpw = n_rows // nw

    mesh = plsc.VectorSubcoreMesh(core_axis_name="c", subcore_axis_name="s")

    @functools.partial(
        pl.kernel,
        mesh=mesh,
        compiler_params=pltpu.CompilerParams(use_tc_tiling_on_sc=False,
                                             needs_layout_passes=False),
        out_type=jax.ShapeDtypeStruct((n_rows * _K * _TOK_W,), jnp.float32),
        scratch_types=[
            pltpu.VMEM((_N,), jnp.float32),
            pltpu.VMEM((_N,), jnp.float32),
            pltpu.VMEM((_N,), jnp.float32),
            pltpu.VMEM((_N,), jnp.float32),
            pltpu.VMEM((_N + 16,), jnp.float32),
            pltpu.VMEM((_SERI + _N + 16,), jnp.int32),
            pltpu.VMEM((_K,), jnp.float32),
            pltpu.VMEM((_K,), jnp.int32),
            pltpu.VMEM((_K, _EMB_PAD), jnp.float32),
            pltpu.VMEM((_K * _TOK_W,), jnp.float32),
            pltpu.VMEM((_K,), jnp.float32),
            pltpu.SemaphoreType.DMA,
            pltpu.SemaphoreType.DMA,
            pltpu.SemaphoreType.DMA,
            pltpu.SemaphoreType.DMA,
            pltpu.SemaphoreType.DMA,
        ],
    )
    def k(rssi_hbm, emb_hbm, rank_hbm, out_hbm,
          row_0, row_1, row_2, row_3, cand_v, cand_i, topv, topi, embbuf,
          tokbuf, rankbuf, sem, rsem_0, rsem_1, rsem_2, rsem_3):
        wid = lax.axis_index("s") * nc + lax.axis_index("c")
        base_row = wid * rpw
        iota16 = lax.iota(jnp.int32, 16)
        pltpu.sync_copy(rank_hbm, rankbuf)

        lane0 = iota16 == 0

        def emit_top(j, m, b):
            jsplat = jnp.full((16,), j, jnp.int32)
            plsc.store_scatter(topv, [jsplat],
                               jnp.broadcast_to(m, (16,)), mask=lane0)
            plsc.store_scatter(topi, [jsplat],
                               jnp.broadcast_to(b, (16,)), mask=lane0)

        def process_row(row, cur, prev, is_t0, t_spec):
            neg1 = jnp.full((16,), -1.0, jnp.float32)

            def run_fast(get_vreg):
                summ = jnp.full((16,), -1.0, jnp.float32)
                for v in range(16):
                    cv, ci, valid = get_vreg(v)
                    cv = jnp.where(valid, cv, -1.0)
                    cand_v[pl.ds(v * 16, 16)] = cv
                    cand_i[pl.ds(_FASTI + v * 16, 16)] = ci
                    summ = jnp.where(iota16 == v, jnp.max(cv), summ)

                def ext(j, summ):
                    m = jnp.max(summ)
                    bv = plsc.all_reduce_ffs(summ == m)[0]
                    sl = pl.ds(bv * 16, 16)
                    cv = cand_v[sl]
                    l0 = plsc.all_reduce_ffs(cv == m)
                    ci = cand_i[pl.ds(_FASTI + bv * 16, 16)]
                    b = ci.at[l0].get(mode="promise_in_bounds")
                    emit_top(j, jnp.full((16,), m), b)
                    cv = jnp.where(iota16 == l0, -1.0, cv)
                    cand_v[sl] = cv
                    return jnp.where(iota16 == bv, jnp.max(cv), summ)

                lax.fori_loop(0, _K, ext, summ)

            def collect_serial(thr):
                @plsc.parallel_loop(0, _NV, unroll=16, carry=jnp.int32(0))
                def collect(c, cnt):
                    msk = cur[pl.ds(c * 16, 16)] >= thr
                    plsc.store_compressed(
                        cand_i.at[pl.ds(_SERI + cnt, 16)],
                        iota16 + c * 16, mask=msk)
                    return cnt + plsc.all_reduce_population_count(msk)[0]

                return collect

            def exact_thr():
                @plsc.parallel_loop(0, _NV // 2, unroll=8, carry=(neg1, neg1))
                def amax(c, ms):
                    m1, m2 = ms
                    return (jnp.maximum(m1, cur[pl.ds(c * 16, 16)]),
                            jnp.maximum(m2,
                                        cur[pl.ds((c + _NV // 2) * 16, 16)]))

                m1, m2 = amax
                return jnp.min(jnp.minimum(m1, m2))

            cnt0 = collect_serial(t_spec)
            fast_ok = (cnt0 >= _K) & (cnt0 <= 256)

            @pl.when(fast_ok)
            def _spec_path():
                def get_spec(v):
                    ci = cand_i[pl.ds(_SERI + v * 16, 16)]
                    valid = iota16 + v * 16 < cnt0
                    cv = plsc.load_gather(cur, [ci], mask=valid)
                    return cv, ci, valid

                run_fast(get_spec)

            @pl.when(jnp.logical_not(fast_ok))
            def _fallback():
                cnt = collect_serial(exact_thr())

                @pl.when(cnt <= 256)
                def _fast():
                    def get_serial(v):
                        ci = cand_i[pl.ds(_SERI + v * 16, 16)]
                        valid = iota16 + v * 16 < cnt
                        cv = plsc.load_gather(cur, [ci], mask=valid)
                        return cv, ci, valid

                    run_fast(get_serial)

                @pl.when(cnt > 256)
                def _slow():
                    nv = (cnt + 15) // 16

                    def mat(v, _):
                        sl = pl.ds(v * 16, 16)
                        valid = iota16 + v * 16 < cnt
                        ci = cand_i[pl.ds(_SERI + v * 16, 16)]
                        cv = plsc.load_gather(cur, [ci], mask=valid)
                        cand_v[sl] = jnp.where(valid, cv, -1.0)
                        return 0

                    lax.fori_loop(0, nv, mat, 0)

                    def extract(j, _):
                        def p1(v, m):
                            return jnp.maximum(m, cand_v[pl.ds(v * 16, 16)])

                        m = jnp.max(lax.fori_loop(0, nv, p1, neg1))

                        def p2(v, b):
                            cv = cand_v[pl.ds(v * 16, 16)]
                            ci = cand_i[pl.ds(_SERI + v * 16, 16)]
                            return jnp.minimum(
                                b, jnp.where(cv == m, ci, _BIG))

                        b = jnp.min(lax.fori_loop(
                            0, nv, p2, jnp.full((16,), _BIG, jnp.int32)))
                        emit_top(j, m, b)

                        def p3(v, _):
                            sl = pl.ds(v * 16, 16)
                            cv = cand_v[sl]
                            ci = cand_i[pl.ds(_SERI + v * 16, 16)]
                            cand_v[sl] = jnp.where(
                                (cv == m) & (ci == b), -1.0, cv)
                            return 0

                        lax.fori_loop(0, nv, p3, 0)
                        return 0

                    lax.fori_loop(0, _K, extract, 0)

            dma = pltpu.async_copy(emb_hbm.at[topi], embbuf, sem)
            for h in range(2):
                sl = pl.ds(h * 16, 16)
                tv = topv[sl]
                ti = topi[sl]
                pv = jnp.where(is_t0, tv, plsc.load_gather(prev, [ti]))
                delta = tv - pv
                isnew = jnp.where((pv <= 1e-6) & (tv > 1e-6), 1.0, 0.0)
                rk = rankbuf[sl]
                tgt = (iota16 + h * 16) * _TOK_W
                plsc.store_scatter(tokbuf, [tgt + 8], tv)
                plsc.store_scatter(tokbuf, [tgt + 9], delta)
                plsc.store_scatter(tokbuf, [tgt + 10], rk)
                plsc.store_scatter(tokbuf, [tgt + 11], isnew)
            dma.wait()
            for h in range(2):
                tok = iota16 + h * 16
                tgt = tok * _TOK_W
                for d in range(8):
                    ev = plsc.load_gather(
                        embbuf, [tok, jnp.full((16,), d, jnp.int32)])
                    plsc.store_scatter(tokbuf, [tgt + d], ev)
            pltpu.sync_copy(tokbuf,
                            out_hbm.at[pl.ds(row * (_K * _TOK_W),
                                             _K * _TOK_W)])
            return 2.0 * topv[pl.ds(_K - 16, 16)][15] - 1.0

        rows = [row_0, row_1, row_2, row_3]
        rsems = [rsem_0, rsem_1, rsem_2, rsem_3]

        def issue_row(r, b):
            rr = jnp.minimum(r, n_rows - 1)
            pltpu.async_copy(rssi_hbm.at[pl.ds(rr * _N, _N)], rows[b],
                             rsems[b])

        def wait_row(b):
            pltpu.make_async_copy(rssi_hbm.at[pl.ds(0, _N)], rows[b],
                                  rsems[b]).wait()

        issue_row(base_row, 0)
        issue_row(base_row + 1, 1)

        def quad(j, t_spec):
            r0 = base_row + 4 * j
            for t in range(4):
                issue_row(r0 + t + 2, (t + 2) % 4)
                wait_row(t)
                t_spec = process_row(r0 + t, rows[t], rows[(t + 3) % 4],
                                     (t == 0) & (j % 8 == 0), t_spec)
            return t_spec

        lax.fori_loop(0, rpw // 4, quad, jnp.float32(2.0))
        wait_row(0)
        wait_row(1)

    return k(rssi_flat, emb_pad, rank_flat)


def kernel(rssi_seq, ap_emb, rank_template):
    b_dim, t_dim, _ = rssi_seq.shape
    emb_dim = ap_emb.shape[1]
    n_rows = b_dim * t_dim
    emb_pad = jnp.pad(ap_emb, ((0, 0), (0, _EMB_PAD - emb_dim)))
    out = _sc_call(rssi_seq.reshape(-1), emb_pad,
                   rank_template.reshape(-1), n_rows)
    return out.reshape(b_dim, t_dim, _K, _TOK_W)
